# baseline (speedup 1.0000x reference)
import functools

import jax
import jax.numpy as jnp
from jax import lax
from jax.experimental import pallas as pl
from jax.experimental.pallas import tpu as pltpu
from jax.experimental.pallas import tpu_sc as plsc

N_EDGES = 320000
N_NODES = 10000
D_FEAT = 128

NC = 2
NS = 16
NW = NC * NS

E_SC = 163840
E_TC = N_EDGES - E_SC

EPT = E_SC // NW
EBLK = 80
NBLK = EPT // EBLK

BE = 512
NBTC = E_TC // BE
TC0 = E_SC // BE
WIN = 256
TCPAD = N_NODES + 496
DEPTH = 4

N_PAD = 10240
RPT = N_PAD // NS
IRT = 624

_mesh = plsc.VectorSubcoreMesh(core_axis_name="c", subcore_axis_name="s")


@functools.partial(
    pl.kernel,
    mesh=_mesh,
    out_type=jax.ShapeDtypeStruct((NC, N_PAD, D_FEAT), jnp.float32),
    scratch_types=(
        [pltpu.VMEM((EBLK,), jnp.int32) for _ in range(DEPTH)]
        + [pltpu.VMEM((EBLK, D_FEAT), jnp.float32) for _ in range(DEPTH)]
        + [pltpu.VMEM_SHARED((N_PAD, D_FEAT), jnp.float32)]
        + [pltpu.SemaphoreType.DMA for _ in range(2 * DEPTH)]
    ),
)
def _sc_scatter_add(src, index, out, part, *refs):
    idxs = refs[0:DEPTH]
    blks = refs[DEPTH:2 * DEPTH]
    acc = refs[2 * DEPTH]
    lsems = refs[2 * DEPTH + 1:3 * DEPTH + 1]
    ssems = refs[3 * DEPTH + 1:4 * DEPTH + 1]
    cid = lax.axis_index("c")
    sid = lax.axis_index("s")
    tid = cid * NS + sid
    base = tid * EPT

    def start_load(b, p):
        off = pl.multiple_of(base + b * EBLK, 8)
        pltpu.make_async_copy(index.at[pl.ds(off, EBLK)], idxs[p],
                              lsems[p]).start()
        pltpu.make_async_copy(src.at[pl.ds(off, EBLK)], blks[p],
                              lsems[p]).start()

    for p in range(DEPTH - 1):
        start_load(p, p)

    @pl.when(sid < NS - 1)
    def _():
        r0 = pl.multiple_of(sid * IRT, 8)
        pltpu.sync_copy(out.at[pl.ds(r0, IRT)], acc.at[pl.ds(r0, IRT)])

    @pl.when(sid == NS - 1)
    def _():
        pltpu.sync_copy(out.at[pl.ds((NS - 1) * IRT, 640)],
                        acc.at[pl.ds((NS - 1) * IRT, 640)])

    plsc.subcore_barrier()

    def wait_load(p):
        pltpu.make_async_copy(index.at[pl.ds(base, EBLK)], idxs[p],
                              lsems[p]).wait()
        pltpu.make_async_copy(src.at[pl.ds(base, EBLK)], blks[p],
                              lsems[p]).wait()

    def start_scat(p):
        pltpu.make_async_copy(blks[p], acc.at[idxs[p]],
                              ssems[p]).start(add=True)

    def wait_scat(p):
        pltpu.make_async_copy(blks[p], acc.at[idxs[p]], ssems[p]).wait()

    def body(i, carry):
        for p in range(DEPTH):
            b = DEPTH * i + p
            wait_load(p)
            start_scat(p)
            q = (p + DEPTH - 1) % DEPTH
            if p == 0:
                @pl.when(i > 0)
                def _():
                    wait_scat(q)
            else:
                wait_scat(q)

            @pl.when(b + DEPTH - 1 < NBLK)
            def _():
                start_load(b + DEPTH - 1, q)
        return carry

    nfull = NBLK // DEPTH
    lax.fori_loop(0, nfull, body, 0)
    for b in range(DEPTH * nfull, NBLK):
        p = b % DEPTH
        wait_load(p)
        start_scat(p)
        wait_scat((p + DEPTH - 1) % DEPTH)
        if b + DEPTH - 1 < NBLK:
            start_load(b + DEPTH - 1, (p + DEPTH - 1) % DEPTH)
    wait_scat((NBLK - 1) % DEPTH)
    plsc.subcore_barrier()

    r0 = pl.multiple_of(sid * RPT, 8)
    pltpu.sync_copy(acc.at[pl.ds(r0, RPT)], part.at[cid, pl.ds(r0, RPT)])


def _tc_seg_body(idx_s, idx_v, src_ref, acc_ref):
    g = pl.program_id(0)

    @pl.when(g == 0)
    def _():
        acc_ref[...] = jnp.zeros_like(acc_ref)

    base = idx_s[0, 0, 0]
    last = idx_s[0, 0, BE - 1]
    base_al = (base // 8) * 8
    rounds = (last - base_al) // WIN + 1
    offs = idx_v[0, 0, :] - base_al
    src = src_ref[...]

    def round_body(r, carry):
        lo = offs - r * WIN
        oh = (lo[:, None] == lax.broadcasted_iota(jnp.int32, (BE, WIN), 1))
        oh = oh.astype(jnp.float32)
        partial = lax.dot_general(oh, src, (((0,), (0,)), ((), ())),
                                  preferred_element_type=jnp.float32)
        row = pl.multiple_of(base_al + r * WIN, 8)
        acc_ref[pl.ds(row, WIN), :] += partial
        return carry

    lax.fori_loop(0, rounds, round_body, 0)


def _tc_segsum(src_tc, idx3):
    return pl.pallas_call(
        _tc_seg_body,
        grid=(NBTC,),
        in_specs=[
            pl.BlockSpec((1, 1, BE), lambda g: (g + TC0, 0, 0),
                         memory_space=pltpu.SMEM),
            pl.BlockSpec((1, 1, BE), lambda g: (g + TC0, 0, 0)),
            pl.BlockSpec((BE, D_FEAT), lambda g: (g + TC0, 0)),
        ],
        out_specs=pl.BlockSpec((TCPAD, D_FEAT), lambda g: (0, 0)),
        out_shape=jax.ShapeDtypeStruct((TCPAD, D_FEAT), jnp.float32),
    )(idx3, idx3, src_tc)


def _combine_body(p_ref, t_ref, o_ref, r_ref):
    r_ref[...] = p_ref[0] + p_ref[1] + t_ref[...] - o_ref[...]


def _combine(part, ptc, out):
    rows = 1000
    return pl.pallas_call(
        _combine_body,
        grid=(N_NODES // rows,),
        in_specs=[
            pl.BlockSpec((NC, rows, D_FEAT), lambda i: (0, i, 0)),
            pl.BlockSpec((rows, D_FEAT), lambda i: (i, 0)),
            pl.BlockSpec((rows, D_FEAT), lambda i: (i, 0)),
        ],
        out_specs=pl.BlockSpec((rows, D_FEAT), lambda i: (i, 0)),
        out_shape=jax.ShapeDtypeStruct((N_NODES, D_FEAT), jnp.float32),
    )(part, ptc, out)


@jax.jit
def kernel(src, index, out):
    idx = index.astype(jnp.int32)
    part = _sc_scatter_add(src, idx, out)
    idx3 = idx.reshape(N_EDGES // BE, 1, BE)
    ptc = _tc_segsum(src, idx3)
    return _combine(part, ptc, out)


# speedup vs baseline: 1.0285x; 1.0285x over previous
import functools

import jax
import jax.numpy as jnp
from jax import lax
from jax.experimental import pallas as pl
from jax.experimental.pallas import tpu as pltpu
from jax.experimental.pallas import tpu_sc as plsc

N_EDGES = 320000
N_NODES = 10000
D_FEAT = 128

NC = 2
NS = 16
NW = NC * NS

E_SC = 163840
E_TC = N_EDGES - E_SC

EPT = E_SC // NW
EBLK = 80
NBLK = EPT // EBLK

BE = 512
NBTC = E_TC // BE
TC0 = E_SC // BE
WIN = 128
TCPAD = N_NODES + 496
DEPTH = 4

N_PAD = 10240
RPT = N_PAD // NS
IRT = 624

_mesh = plsc.VectorSubcoreMesh(core_axis_name="c", subcore_axis_name="s")


@functools.partial(
    pl.kernel,
    mesh=_mesh,
    out_type=jax.ShapeDtypeStruct((NC, N_PAD, D_FEAT), jnp.float32),
    scratch_types=(
        [pltpu.VMEM((EBLK,), jnp.int32) for _ in range(DEPTH)]
        + [pltpu.VMEM((EBLK, D_FEAT), jnp.float32) for _ in range(DEPTH)]
        + [pltpu.VMEM_SHARED((N_PAD, D_FEAT), jnp.float32)]
        + [pltpu.SemaphoreType.DMA for _ in range(2 * DEPTH)]
    ),
)
def _sc_scatter_add(src, index, out, part, *refs):
    idxs = refs[0:DEPTH]
    blks = refs[DEPTH:2 * DEPTH]
    acc = refs[2 * DEPTH]
    lsems = refs[2 * DEPTH + 1:3 * DEPTH + 1]
    ssems = refs[3 * DEPTH + 1:4 * DEPTH + 1]
    cid = lax.axis_index("c")
    sid = lax.axis_index("s")
    tid = cid * NS + sid
    base = tid * EPT

    def start_load(b, p):
        off = pl.multiple_of(base + b * EBLK, 8)
        pltpu.make_async_copy(index.at[pl.ds(off, EBLK)], idxs[p],
                              lsems[p]).start()
        pltpu.make_async_copy(src.at[pl.ds(off, EBLK)], blks[p],
                              lsems[p]).start()

    for p in range(DEPTH - 1):
        start_load(p, p)

    @pl.when(sid < NS - 1)
    def _():
        r0 = pl.multiple_of(sid * IRT, 8)
        pltpu.sync_copy(out.at[pl.ds(r0, IRT)], acc.at[pl.ds(r0, IRT)])

    @pl.when(sid == NS - 1)
    def _():
        pltpu.sync_copy(out.at[pl.ds((NS - 1) * IRT, 640)],
                        acc.at[pl.ds((NS - 1) * IRT, 640)])

    plsc.subcore_barrier()

    def wait_load(p):
        pltpu.make_async_copy(index.at[pl.ds(base, EBLK)], idxs[p],
                              lsems[p]).wait()
        pltpu.make_async_copy(src.at[pl.ds(base, EBLK)], blks[p],
                              lsems[p]).wait()

    def start_scat(p):
        pltpu.make_async_copy(blks[p], acc.at[idxs[p]],
                              ssems[p]).start(add=True)

    def wait_scat(p):
        pltpu.make_async_copy(blks[p], acc.at[idxs[p]], ssems[p]).wait()

    def body(i, carry):
        for p in range(DEPTH):
            b = DEPTH * i + p
            wait_load(p)
            start_scat(p)
            q = (p + DEPTH - 1) % DEPTH
            if p == 0:
                @pl.when(i > 0)
                def _():
                    wait_scat(q)
            else:
                wait_scat(q)

            @pl.when(b + DEPTH - 1 < NBLK)
            def _():
                start_load(b + DEPTH - 1, q)
        return carry

    nfull = NBLK // DEPTH
    lax.fori_loop(0, nfull, body, 0)
    for b in range(DEPTH * nfull, NBLK):
        p = b % DEPTH
        wait_load(p)
        start_scat(p)
        wait_scat((p + DEPTH - 1) % DEPTH)
        if b + DEPTH - 1 < NBLK:
            start_load(b + DEPTH - 1, (p + DEPTH - 1) % DEPTH)
    wait_scat((NBLK - 1) % DEPTH)
    plsc.subcore_barrier()

    r0 = pl.multiple_of(sid * RPT, 8)
    pltpu.sync_copy(acc.at[pl.ds(r0, RPT)], part.at[cid, pl.ds(r0, RPT)])


def _tc_seg_body(idx_s, idx_v, src_ref, acc_ref):
    g = pl.program_id(0)

    @pl.when(g == 0)
    def _():
        acc_ref[...] = jnp.zeros_like(acc_ref)

    base = idx_s[0, 0, 0]
    last = idx_s[0, 0, BE - 1]
    base_al = (base // 8) * 8
    rounds = (last - base_al) // WIN + 1
    offs = idx_v[0, 0, :] - base_al
    src = src_ref[...]

    def round_body(r, carry):
        lo = offs - r * WIN
        oh = (lo[:, None] == lax.broadcasted_iota(jnp.int32, (BE, WIN), 1))
        oh = oh.astype(jnp.float32)
        partial = lax.dot_general(oh, src, (((0,), (0,)), ((), ())),
                                  preferred_element_type=jnp.float32)
        row = pl.multiple_of(base_al + r * WIN, 8)
        acc_ref[pl.ds(row, WIN), :] += partial
        return carry

    lax.fori_loop(0, rounds, round_body, 0)


def _tc_segsum(src_tc, idx3):
    return pl.pallas_call(
        _tc_seg_body,
        grid=(NBTC,),
        in_specs=[
            pl.BlockSpec((1, 1, BE), lambda g: (g + TC0, 0, 0),
                         memory_space=pltpu.SMEM),
            pl.BlockSpec((1, 1, BE), lambda g: (g + TC0, 0, 0)),
            pl.BlockSpec((BE, D_FEAT), lambda g: (g + TC0, 0)),
        ],
        out_specs=pl.BlockSpec((TCPAD, D_FEAT), lambda g: (0, 0)),
        out_shape=jax.ShapeDtypeStruct((TCPAD, D_FEAT), jnp.float32),
    )(idx3, idx3, src_tc)


def _combine_body(p_ref, t_ref, o_ref, r_ref):
    r_ref[...] = p_ref[0] + p_ref[1] + t_ref[...] - o_ref[...]


def _combine(part, ptc, out):
    rows = 1000
    return pl.pallas_call(
        _combine_body,
        grid=(N_NODES // rows,),
        in_specs=[
            pl.BlockSpec((NC, rows, D_FEAT), lambda i: (0, i, 0)),
            pl.BlockSpec((rows, D_FEAT), lambda i: (i, 0)),
            pl.BlockSpec((rows, D_FEAT), lambda i: (i, 0)),
        ],
        out_specs=pl.BlockSpec((rows, D_FEAT), lambda i: (i, 0)),
        out_shape=jax.ShapeDtypeStruct((N_NODES, D_FEAT), jnp.float32),
    )(part, ptc, out)


@jax.jit
def kernel(src, index, out):
    idx = index.astype(jnp.int32)
    part = _sc_scatter_add(src, idx, out)
    idx3 = idx.reshape(N_EDGES // BE, 1, BE)
    ptc = _tc_segsum(src, idx3)
    return _combine(part, ptc, out)


# speedup vs baseline: 1.9959x; 1.9406x over previous
import functools

import jax
import jax.numpy as jnp
from jax import lax
from jax.experimental import pallas as pl
from jax.experimental.pallas import tpu as pltpu
from jax.experimental.pallas import tpu_sc as plsc

N_EDGES = 320000
N_NODES = 10000
D_FEAT = 128

NC = 2
NS = 16
NW = NC * NS

EPT = N_EDGES // NW
EBLK = 80
NBLK = EPT // EBLK
DEPTH = 4

N_PAD = 10240
RPT = N_PAD // NS
IRT = 624

_mesh = plsc.VectorSubcoreMesh(core_axis_name="c", subcore_axis_name="s")


@functools.partial(
    pl.kernel,
    mesh=_mesh,
    out_type=jax.ShapeDtypeStruct((NC, N_PAD, D_FEAT), jnp.float32),
    scratch_types=(
        [pltpu.VMEM((EBLK,), jnp.int32) for _ in range(DEPTH)]
        + [pltpu.VMEM((EBLK, D_FEAT), jnp.float32) for _ in range(DEPTH)]
        + [pltpu.VMEM_SHARED((N_PAD, D_FEAT), jnp.float32)]
        + [pltpu.SemaphoreType.DMA for _ in range(2 * DEPTH)]
    ),
)
def _sc_scatter_add(src, index, out, part, *refs):
    idxs = refs[0:DEPTH]
    blks = refs[DEPTH:2 * DEPTH]
    acc = refs[2 * DEPTH]
    lsems = refs[2 * DEPTH + 1:3 * DEPTH + 1]
    ssems = refs[3 * DEPTH + 1:4 * DEPTH + 1]
    cid = lax.axis_index("c")
    sid = lax.axis_index("s")
    tid = cid * NS + sid
    base = tid * EPT

    def start_load(b, p):
        off = pl.multiple_of(base + b * EBLK, 8)
        pltpu.make_async_copy(index.at[pl.ds(off, EBLK)], idxs[p],
                              lsems[p]).start()
        pltpu.make_async_copy(src.at[pl.ds(off, EBLK)], blks[p],
                              lsems[p]).start()

    for p in range(DEPTH - 1):
        start_load(p, p)

    @pl.when(sid < NS - 1)
    def _():
        r0 = pl.multiple_of(sid * IRT, 8)
        pltpu.sync_copy(out.at[pl.ds(r0, IRT)], acc.at[pl.ds(r0, IRT)])

    @pl.when(sid == NS - 1)
    def _():
        pltpu.sync_copy(out.at[pl.ds((NS - 1) * IRT, 640)],
                        acc.at[pl.ds((NS - 1) * IRT, 640)])

    plsc.subcore_barrier()

    def wait_load(p):
        pltpu.make_async_copy(index.at[pl.ds(base, EBLK)], idxs[p],
                              lsems[p]).wait()
        pltpu.make_async_copy(src.at[pl.ds(base, EBLK)], blks[p],
                              lsems[p]).wait()

    def start_scat(p):
        pltpu.make_async_copy(blks[p], acc.at[idxs[p]],
                              ssems[p]).start(add=True)

    def wait_scat(p):
        pltpu.make_async_copy(blks[p], acc.at[idxs[p]], ssems[p]).wait()

    def body(i, carry):
        for p in range(DEPTH):
            b = DEPTH * i + p
            wait_load(p)
            start_scat(p)
            q = (p + DEPTH - 1) % DEPTH
            if p == 0:
                @pl.when(i > 0)
                def _():
                    wait_scat(q)
            else:
                wait_scat(q)

            @pl.when(b + DEPTH - 1 < NBLK)
            def _():
                start_load(b + DEPTH - 1, q)
        return carry

    nfull = NBLK // DEPTH
    lax.fori_loop(0, nfull, body, 0)
    for b in range(DEPTH * nfull, NBLK):
        p = b % DEPTH
        wait_load(p)
        start_scat(p)
        wait_scat((p + DEPTH - 1) % DEPTH)
        if b + DEPTH - 1 < NBLK:
            start_load(b + DEPTH - 1, (p + DEPTH - 1) % DEPTH)
    wait_scat((NBLK - 1) % DEPTH)
    plsc.subcore_barrier()

    r0 = pl.multiple_of(sid * RPT, 8)
    pltpu.sync_copy(acc.at[pl.ds(r0, RPT)], part.at[cid, pl.ds(r0, RPT)])


def _combine_body(p_ref, o_ref, r_ref):
    r_ref[...] = p_ref[0] + p_ref[1] - o_ref[...]


def _combine(part, out):
    rows = 1000
    return pl.pallas_call(
        _combine_body,
        grid=(N_NODES // rows,),
        in_specs=[
            pl.BlockSpec((NC, rows, D_FEAT), lambda i: (0, i, 0)),
            pl.BlockSpec((rows, D_FEAT), lambda i: (i, 0)),
        ],
        out_specs=pl.BlockSpec((rows, D_FEAT), lambda i: (i, 0)),
        out_shape=jax.ShapeDtypeStruct((N_NODES, D_FEAT), jnp.float32),
    )(part, out)


@jax.jit
def kernel(src, index, out):
    part = _sc_scatter_add(src, index.astype(jnp.int32), out)
    return _combine(part, out)


# speedup vs baseline: 2.0233x; 1.0137x over previous
import functools

import jax
import jax.numpy as jnp
from jax import lax
from jax.experimental import pallas as pl
from jax.experimental.pallas import tpu as pltpu
from jax.experimental.pallas import tpu_sc as plsc

N_EDGES = 320000
N_NODES = 10000
D_FEAT = 128

NC = 2
NS = 16
NW = NC * NS

EPT = N_EDGES // NW
EBLK = 80
NBLK = EPT // EBLK
DEPTH = 4

N_PAD = 10240
RPT = N_PAD // NS
IRT = 624

_mesh = plsc.VectorSubcoreMesh(core_axis_name="c", subcore_axis_name="s")


@functools.partial(
    pl.kernel,
    mesh=_mesh,
    out_type=jax.ShapeDtypeStruct((NC, N_PAD, D_FEAT), jnp.float32),
    scratch_types=(
        [pltpu.VMEM((EBLK,), jnp.int32) for _ in range(DEPTH)]
        + [pltpu.VMEM((EBLK, D_FEAT), jnp.float32) for _ in range(DEPTH)]
        + [pltpu.VMEM_SHARED((N_PAD, D_FEAT), jnp.float32)]
        + [pltpu.SemaphoreType.DMA for _ in range(2 * DEPTH)]
    ),
)
def _sc_scatter_add(src, index, out, part, *refs):
    idxs = refs[0:DEPTH]
    blks = refs[DEPTH:2 * DEPTH]
    acc = refs[2 * DEPTH]
    lsems = refs[2 * DEPTH + 1:3 * DEPTH + 1]
    ssems = refs[3 * DEPTH + 1:4 * DEPTH + 1]
    cid = lax.axis_index("c")
    sid = lax.axis_index("s")
    tid = cid * NS + sid
    base = tid * EPT

    def start_load(b, p):
        off = pl.multiple_of(base + b * EBLK, 8)
        pltpu.make_async_copy(index.at[pl.ds(off, EBLK)], idxs[p],
                              lsems[p]).start()
        pltpu.make_async_copy(src.at[pl.ds(off, EBLK)], blks[p],
                              lsems[p]).start()

    for p in range(DEPTH - 1):
        start_load(p, p)

    @pl.when(sid < NS - 1)
    def _():
        r0 = pl.multiple_of(sid * IRT, 8)
        pltpu.sync_copy(out.at[pl.ds(r0, IRT)], acc.at[pl.ds(r0, IRT)])

    @pl.when(sid == NS - 1)
    def _():
        pltpu.sync_copy(out.at[pl.ds((NS - 1) * IRT, 640)],
                        acc.at[pl.ds((NS - 1) * IRT, 640)])

    plsc.subcore_barrier()

    def wait_load(p):
        pltpu.make_async_copy(index.at[pl.ds(base, EBLK)], idxs[p],
                              lsems[p]).wait()
        pltpu.make_async_copy(src.at[pl.ds(base, EBLK)], blks[p],
                              lsems[p]).wait()

    def start_scat(p):
        pltpu.make_async_copy(blks[p], acc.at[idxs[p]],
                              ssems[p]).start(add=True)

    def wait_scat(p):
        pltpu.make_async_copy(blks[p], acc.at[idxs[p]], ssems[p]).wait()

    def body(i, carry):
        for p in range(DEPTH):
            b = DEPTH * i + p
            wait_load(p)
            start_scat(p)
            q = (p + DEPTH - 1) % DEPTH
            if p == 0:
                @pl.when(i > 0)
                def _():
                    wait_scat(q)
            else:
                wait_scat(q)

            @pl.when(b + DEPTH - 1 < NBLK)
            def _():
                start_load(b + DEPTH - 1, q)
        return carry

    nfull = NBLK // DEPTH
    lax.fori_loop(0, nfull, body, 0)
    for b in range(DEPTH * nfull, NBLK):
        p = b % DEPTH
        wait_load(p)
        start_scat(p)
        wait_scat((p + DEPTH - 1) % DEPTH)
        if b + DEPTH - 1 < NBLK:
            start_load(b + DEPTH - 1, (p + DEPTH - 1) % DEPTH)
    wait_scat((NBLK - 1) % DEPTH)
    plsc.subcore_barrier()

    r0 = pl.multiple_of(sid * RPT, 8)
    pltpu.sync_copy(acc.at[pl.ds(r0, RPT)], part.at[cid, pl.ds(r0, RPT)])


def _combine_body(p_ref, o_ref, r_ref):
    r_ref[...] = p_ref[0] + p_ref[1] - o_ref[...]


def _combine(part, out):
    rows = 2000
    return pl.pallas_call(
        _combine_body,
        grid=(N_NODES // rows,),
        in_specs=[
            pl.BlockSpec((NC, rows, D_FEAT), lambda i: (0, i, 0)),
            pl.BlockSpec((rows, D_FEAT), lambda i: (i, 0)),
        ],
        out_specs=pl.BlockSpec((rows, D_FEAT), lambda i: (i, 0)),
        out_shape=jax.ShapeDtypeStruct((N_NODES, D_FEAT), jnp.float32),
    )(part, out)


@jax.jit
def kernel(src, index, out):
    part = _sc_scatter_add(src, index.astype(jnp.int32), out)
    return _combine(part, out)


# speedup vs baseline: 2.0346x; 1.0056x over previous
import functools

import jax
import jax.numpy as jnp
from jax import lax
from jax.experimental import pallas as pl
from jax.experimental.pallas import tpu as pltpu
from jax.experimental.pallas import tpu_sc as plsc

N_EDGES = 320000
N_NODES = 10000
D_FEAT = 128

NC = 2
NS = 16
NW = NC * NS

EPT = N_EDGES // NW
EBLK = 80
NBLK = EPT // EBLK
DEPTH = 4

N_PAD = 10240
RPT = N_PAD // NS
IRT = 624

_mesh = plsc.VectorSubcoreMesh(core_axis_name="c", subcore_axis_name="s")


@functools.partial(
    pl.kernel,
    mesh=_mesh,
    out_type=jax.ShapeDtypeStruct((NC, N_PAD, D_FEAT), jnp.float32),
    scratch_types=(
        [pltpu.VMEM((EBLK,), jnp.int32) for _ in range(DEPTH)]
        + [pltpu.VMEM((EBLK, D_FEAT), jnp.float32) for _ in range(DEPTH)]
        + [pltpu.VMEM_SHARED((N_PAD, D_FEAT), jnp.float32)]
        + [pltpu.SemaphoreType.DMA for _ in range(2 * DEPTH)]
    ),
)
def _sc_scatter_add(src, index, out, part, *refs):
    idxs = refs[0:DEPTH]
    blks = refs[DEPTH:2 * DEPTH]
    acc = refs[2 * DEPTH]
    lsems = refs[2 * DEPTH + 1:3 * DEPTH + 1]
    ssems = refs[3 * DEPTH + 1:4 * DEPTH + 1]
    cid = lax.axis_index("c")
    sid = lax.axis_index("s")
    tid = cid * NS + sid
    base = tid * EPT

    def start_load(b, p):
        off = pl.multiple_of(base + b * EBLK, 8)
        pltpu.make_async_copy(index.at[pl.ds(off, EBLK)], idxs[p],
                              lsems[p]).start()
        pltpu.make_async_copy(src.at[pl.ds(off, EBLK)], blks[p],
                              lsems[p]).start()

    for p in range(DEPTH - 1):
        start_load(p, p)

    @pl.when(sid < NS - 1)
    def _():
        r0 = pl.multiple_of(sid * IRT, 8)
        pltpu.sync_copy(out.at[pl.ds(r0, IRT)], acc.at[pl.ds(r0, IRT)])

    @pl.when(sid == NS - 1)
    def _():
        pltpu.sync_copy(out.at[pl.ds((NS - 1) * IRT, 640)],
                        acc.at[pl.ds((NS - 1) * IRT, 640)])

    plsc.subcore_barrier()

    def wait_load(p):
        pltpu.make_async_copy(index.at[pl.ds(base, EBLK)], idxs[p],
                              lsems[p]).wait()
        pltpu.make_async_copy(src.at[pl.ds(base, EBLK)], blks[p],
                              lsems[p]).wait()

    def start_scat(p):
        pltpu.make_async_copy(blks[p], acc.at[idxs[p]],
                              ssems[p]).start(add=True)

    def wait_scat(p):
        pltpu.make_async_copy(blks[p], acc.at[idxs[p]], ssems[p]).wait()

    def body(i, carry):
        for p in range(DEPTH):
            b = DEPTH * i + p
            wait_load(p)
            start_scat(p)
            q = (p + DEPTH - 1) % DEPTH
            if p == 0:
                @pl.when(i > 0)
                def _():
                    wait_scat(q)
            else:
                wait_scat(q)

            @pl.when(b + DEPTH - 1 < NBLK)
            def _():
                start_load(b + DEPTH - 1, q)
        return carry

    nfull = NBLK // DEPTH
    lax.fori_loop(0, nfull, body, 0)
    for b in range(DEPTH * nfull, NBLK):
        p = b % DEPTH
        wait_load(p)
        start_scat(p)
        wait_scat((p + DEPTH - 1) % DEPTH)
        if b + DEPTH - 1 < NBLK:
            start_load(b + DEPTH - 1, (p + DEPTH - 1) % DEPTH)
    wait_scat((NBLK - 1) % DEPTH)
    plsc.subcore_barrier()

    r0 = pl.multiple_of(sid * RPT, 8)
    pltpu.sync_copy(acc.at[pl.ds(r0, RPT)], part.at[cid, pl.ds(r0, RPT)])


def _combine_body(p_ref, o_ref, r_ref):
    r_ref[...] = p_ref[0] + p_ref[1] - o_ref[...]


def _combine(part, out):
    rows = 5000
    return pl.pallas_call(
        _combine_body,
        grid=(N_NODES // rows,),
        in_specs=[
            pl.BlockSpec((NC, rows, D_FEAT), lambda i: (0, i, 0)),
            pl.BlockSpec((rows, D_FEAT), lambda i: (i, 0)),
        ],
        out_specs=pl.BlockSpec((rows, D_FEAT), lambda i: (i, 0)),
        out_shape=jax.ShapeDtypeStruct((N_NODES, D_FEAT), jnp.float32),
    )(part, out)


@jax.jit
def kernel(src, index, out):
    part = _sc_scatter_add(src, index.astype(jnp.int32), out)
    return _combine(part, out)
